# baseline (speedup 1.0000x reference)
import functools

import jax
import jax.numpy as jnp
from jax import lax
from jax.experimental import pallas as pl
from jax.experimental.pallas import tpu as pltpu
from jax.experimental.pallas import tpu_sc as plsc

_F32 = jnp.float32


def _dot(a, b):
    return jnp.dot(a, b, preferred_element_type=_F32)


def _dot_hi(a, b):
    return jnp.dot(a, b, preferred_element_type=_F32,
                   precision=jax.lax.Precision.HIGHEST)


def _mm_body(x_ref, w_ref, o_ref):
    o_ref[...] = _dot(x_ref[...], w_ref[...])


def _matmul(x, w, bn):
    n, k = x.shape
    m = w.shape[1]
    return pl.pallas_call(
        _mm_body,
        grid=(n // bn,),
        in_specs=[pl.BlockSpec((bn, k), lambda i: (i, 0)),
                  pl.BlockSpec((k, m), lambda i: (0, 0))],
        out_specs=pl.BlockSpec((bn, m), lambda i: (i, 0)),
        out_shape=jax.ShapeDtypeStruct((n, m), _F32),
    )(x, w)


def _enc_body(ea_ref, w_ref, b_ref, o_ref):
    o_ref[...] = _dot(ea_ref[...], w_ref[...]) + b_ref[...]


def _edge_enc(edge_attr, w_t, b, be):
    e_num, d = edge_attr.shape
    h = w_t.shape[1]
    return pl.pallas_call(
        _enc_body,
        grid=(e_num // be,),
        in_specs=[pl.BlockSpec((be, d), lambda i: (i, 0)),
                  pl.BlockSpec((d, h), lambda i: (0, 0)),
                  pl.BlockSpec((1, h), lambda i: (0, 0))],
        out_specs=pl.BlockSpec((be, h), lambda i: (i, 0)),
        out_shape=jax.ShapeDtypeStruct((e_num, h), _F32),
    )(edge_attr, w_t, b)


def _pass_a_body(xr_ref, xc_ref, e_ref, w1rc_ref, we1_ref, b1_ref, w2_ref,
                 b2_ref, wn_ref, bn_ref, wn2_ref, bn2_ref, eo_ref, m_ref):
    xr = xr_ref[...]
    cat_rc = jnp.concatenate([xr, xc_ref[...]], axis=1)
    h1 = jnp.maximum(_dot(cat_rc, w1rc_ref[...])
                     + _dot(e_ref[...], we1_ref[...]) + b1_ref[...], 0.0)
    e2 = _dot(h1, w2_ref[...]) + b2_ref[...]
    eo_ref[...] = e2
    cat_n = jnp.concatenate([xr, e2], axis=1)
    h2 = jnp.maximum(_dot(cat_n, wn_ref[...]) + bn_ref[...], 0.0)
    m_ref[...] = _dot(h2, wn2_ref[...]) + bn2_ref[...]


def _pass_a(xr, xc, e, w1rc_t, we1_t, b1, w2_t, b2, wn_t, bn, wn2_t, bn2,
            be):
    e_num, h = e.shape
    kern = pl.pallas_call(
        _pass_a_body,
        grid=(e_num // be,),
        in_specs=[pl.BlockSpec((be, h), lambda i: (i, 0)),
                  pl.BlockSpec((be, h), lambda i: (i, 0)),
                  pl.BlockSpec((be, h), lambda i: (i, 0)),
                  pl.BlockSpec((2 * h, h), lambda i: (0, 0)),
                  pl.BlockSpec((h, h), lambda i: (0, 0)),
                  pl.BlockSpec((1, h), lambda i: (0, 0)),
                  pl.BlockSpec((h, h), lambda i: (0, 0)),
                  pl.BlockSpec((1, h), lambda i: (0, 0)),
                  pl.BlockSpec((2 * h, h), lambda i: (0, 0)),
                  pl.BlockSpec((1, h), lambda i: (0, 0)),
                  pl.BlockSpec((h, h), lambda i: (0, 0)),
                  pl.BlockSpec((1, h), lambda i: (0, 0))],
        out_specs=[pl.BlockSpec((be, h), lambda i: (i, 0)),
                   pl.BlockSpec((be, h), lambda i: (i, 0))],
        out_shape=[jax.ShapeDtypeStruct((e_num, h), _F32),
                   jax.ShapeDtypeStruct((e_num, h), _F32)],
    )
    return kern(xr, xc, e, w1rc_t, we1_t, b1, w2_t, b2, wn_t, bn, wn2_t, bn2)


def _pass_b_body(gd_ref, e_ref, w_ref, b_ref, w2_ref, b2_ref, o_ref):
    cat = jnp.concatenate([gd_ref[...], e_ref[...]], axis=1)
    h4 = jnp.maximum(_dot(cat, w_ref[...]) + b_ref[...], 0.0)
    o_ref[...] = _dot(h4, w2_ref[...]) + b2_ref[...]


def _pass_b(gd, e, w_t, b, w2_t, b2, be):
    e_num, h = e.shape
    return pl.pallas_call(
        _pass_b_body,
        grid=(e_num // be,),
        in_specs=[pl.BlockSpec((be, h), lambda i: (i, 0)),
                  pl.BlockSpec((be, h), lambda i: (i, 0)),
                  pl.BlockSpec((2 * h, h), lambda i: (0, 0)),
                  pl.BlockSpec((1, h), lambda i: (0, 0)),
                  pl.BlockSpec((h, h), lambda i: (0, 0)),
                  pl.BlockSpec((1, h), lambda i: (0, 0))],
        out_specs=pl.BlockSpec((be, h), lambda i: (i, 0)),
        out_shape=jax.ShapeDtypeStruct((e_num, h), _F32),
    )(gd, e, w_t, b, w2_t, b2)


def _pass_c_body(ge_ref, gf_ref, e_ref, w1rc_ref, we1_ref, b1_ref, w2_ref,
                 b2_ref, o_ref):
    cat = jnp.concatenate([ge_ref[...], gf_ref[...]], axis=1)
    h1 = jnp.maximum(_dot(cat, w1rc_ref[...])
                     + _dot(e_ref[...], we1_ref[...]) + b1_ref[...], 0.0)
    o_ref[...] = _dot(h1, w2_ref[...]) + b2_ref[...]


def _pass_c(ge, gf, e, w1rc_t, we1_t, b1, w2_t, b2, be):
    e_num, h = e.shape
    return pl.pallas_call(
        _pass_c_body,
        grid=(e_num // be,),
        in_specs=[pl.BlockSpec((be, h), lambda i: (i, 0)),
                  pl.BlockSpec((be, h), lambda i: (i, 0)),
                  pl.BlockSpec((be, h), lambda i: (i, 0)),
                  pl.BlockSpec((2 * h, h), lambda i: (0, 0)),
                  pl.BlockSpec((h, h), lambda i: (0, 0)),
                  pl.BlockSpec((1, h), lambda i: (0, 0)),
                  pl.BlockSpec((h, h), lambda i: (0, 0)),
                  pl.BlockSpec((1, h), lambda i: (0, 0))],
        out_specs=pl.BlockSpec((be, h), lambda i: (i, 0)),
        out_shape=jax.ShapeDtypeStruct((e_num, h), _F32),
    )(ge, gf, e, w1rc_t, we1_t, b1, w2_t, b2)


def _pool_body(acc_ref, batch_ref, w1_ref, b1_ref, w2_ref, b2_ref,
               o_ref, sacc_ref, *, g_num, n_blocks):
    i = pl.program_id(0)

    @pl.when(i == 0)
    def _init():
        sacc_ref[...] = jnp.zeros_like(sacc_ref)

    bvals = batch_ref[0, 0, :]
    pad = sacc_ref.shape[0]
    oh = (bvals[:, None] == jax.lax.broadcasted_iota(jnp.int32, (1, pad), 1)
          ).astype(_F32)
    sacc_ref[...] += _dot_hi(oh.T, acc_ref[...])

    @pl.when(i == n_blocks - 1)
    def _final():
        hid = jnp.maximum(_dot(sacc_ref[:g_num], w1_ref[...]) + b1_ref[...],
                          0.0)
        o_ref[...] = _dot(hid, w2_ref[...]) + b2_ref[...]


def _pool_head(acc, batch, w1_t, b1, w2_t, b2, g_num, bn):
    n, h = acc.shape
    n_blocks = n // bn
    pad = 128
    batch3d = batch.reshape(n_blocks, 1, bn)
    body = functools.partial(_pool_body, g_num=g_num, n_blocks=n_blocks)
    return pl.pallas_call(
        body,
        grid=(n_blocks,),
        in_specs=[pl.BlockSpec((bn, h), lambda i: (i, 0)),
                  pl.BlockSpec((1, 1, bn), lambda i: (i, 0, 0)),
                  pl.BlockSpec((h, h), lambda i: (0, 0)),
                  pl.BlockSpec((1, h), lambda i: (0, 0)),
                  pl.BlockSpec((h, 1), lambda i: (0, 0)),
                  pl.BlockSpec((1, 1), lambda i: (0, 0))],
        out_specs=pl.BlockSpec((g_num, 1), lambda i: (0, 0)),
        out_shape=jax.ShapeDtypeStruct((g_num, 1), _F32),
        scratch_shapes=[pltpu.VMEM((pad, h), _F32)],
    )(acc, batch3d, w1_t, b1, w2_t, b2)


_NW = 32
_CH = 40
_KB = 5


def _sc_gather_multi(tables, idxs):
    e_num = idxs[0].shape[0]
    ew = e_num // _NW
    nchunks = ew // _CH
    ngroups = nchunks // _KB
    assert e_num % _NW == 0 and ew % (_CH * _KB) == 0
    nt = len(tables)
    widths = [t.shape[1] for t in tables]

    scratch = []
    for w in widths:
        scratch.append(pltpu.VMEM((ew,), jnp.int32))
        scratch.append(pltpu.VMEM((_KB, _CH, w), _F32))
        for _ in range(2 * _KB):
            scratch.append(pltpu.SemaphoreType.DMA)

    mesh = plsc.VectorSubcoreMesh(core_axis_name="c", subcore_axis_name="s")

    @functools.partial(
        pl.kernel, mesh=mesh,
        out_type=[jax.ShapeDtypeStruct((e_num, w), _F32) for w in widths],
        scratch_types=scratch,
    )
    def gather_kernel(*args):
        tabs = args[:nt]
        ixs = args[nt:2 * nt]
        outs = args[2 * nt:3 * nt]
        rest = list(args[3 * nt:])
        wid = lax.axis_index("s") * 2 + lax.axis_index("c")
        base = wid * ew

        for t in range(nt):
            idx_v = rest[0]
            bufs = rest[1]
            gsems = rest[2:2 + _KB]
            wsems = rest[2 + _KB:2 + 2 * _KB]
            rest = rest[2 + 2 * _KB:]
            tab, out = tabs[t], outs[t]

            pltpu.sync_copy(ixs[t].at[pl.ds(base, ew)], idx_v)

            def group(g, _, tab=tab, out=out, idx_v=idx_v, bufs=bufs,
                      gsems=gsems, wsems=wsems):
                for b in range(_KB):
                    j = g * _KB + b

                    @pl.when(g > 0)
                    def _free():
                        pltpu.make_async_copy(
                            bufs.at[b], out.at[pl.ds(base, _CH)],
                            wsems[b]).wait()

                    pltpu.async_copy(tab.at[idx_v.at[pl.ds(j * _CH, _CH)]],
                                     bufs.at[b], gsems[b])
                for b in range(_KB):
                    j = g * _KB + b
                    pltpu.make_async_copy(
                        tab.at[idx_v.at[pl.ds(j * _CH, _CH)]], bufs.at[b],
                        gsems[b]).wait()
                    pltpu.async_copy(bufs.at[b],
                                     out.at[pl.ds(base + j * _CH, _CH)],
                                     wsems[b])
                return 0

            lax.fori_loop(0, ngroups, group, 0)
            for b in range(_KB):
                pltpu.make_async_copy(bufs.at[b], out.at[pl.ds(base, _CH)],
                                      wsems[b]).wait()

    return gather_kernel(*tables, *idxs)


def _scatter_add(vals, idx, n):
    return jax.ops.segment_sum(vals, idx, num_segments=n)


def kernel(x, edge_attr, edge_index, batch,
           edge_enc_W, edge_enc_b,
           etn_edge_W1, etn_edge_b1, etn_edge_W2, etn_edge_b2,
           etn_node_W1, etn_node_b1, etn_node_W2, etn_node_b2,
           nte_node_W1, nte_node_b1, nte_node_W2, nte_node_b2,
           nte_edge_W1, nte_edge_b1, nte_edge_W2, nte_edge_b2,
           reg_W1, reg_b1, reg_W2, reg_b2):
    n, h = x.shape
    e_num = edge_attr.shape[0]
    n_layers = etn_edge_W1.shape[0]
    g_num = 64
    be = 2000
    bn = 2000
    row = edge_index[0]
    col = edge_index[1]

    rvec = lambda v: v.reshape(1, -1)

    e = _edge_enc(edge_attr, edge_enc_W.T, rvec(edge_enc_b), be)

    acc = x
    xr = xc = None
    for l in range(n_layers):
        if xr is None:
            xr, xc = _sc_gather_multi([acc, acc], [row, col])
        e, m2 = _pass_a(xr, xc, e,
                        etn_edge_W1[l, :, :2 * h].T,
                        etn_edge_W1[l, :, 2 * h:].T, rvec(etn_edge_b1[l]),
                        etn_edge_W2[l].T, rvec(etn_edge_b2[l]),
                        etn_node_W1[l].T, rvec(etn_node_b1[l]),
                        etn_node_W2[l].T, rvec(etn_node_b2[l]), be)
        acc = _scatter_add(m2, col, n)

        (gd,) = _sc_gather_multi([acc], [row])
        m4 = _pass_b(gd, e, nte_node_W1[l].T, rvec(nte_node_b1[l]),
                     nte_node_W2[l].T, rvec(nte_node_b2[l]), be)
        if l < n_layers - 1:
            acc = _scatter_add(m4, col, n)
            xr, xc = _sc_gather_multi([acc, acc], [row, col])
            e = _pass_c(xr, xc, e,
                        nte_edge_W1[l, :, :2 * h].T,
                        nte_edge_W1[l, :, 2 * h:].T, rvec(nte_edge_b1[l]),
                        nte_edge_W2[l].T, rvec(nte_edge_b2[l]), be)

    bc = jnp.take(batch, col)
    out = _pool_head(m4, bc, reg_W1.T, rvec(reg_b1),
                     reg_W2.T, rvec(reg_b2), g_num, be)
    return out.reshape(-1)


# speedup vs baseline: 1.1976x; 1.1976x over previous
import functools

import jax
import jax.numpy as jnp
from jax import lax
from jax.experimental import pallas as pl
from jax.experimental.pallas import tpu as pltpu
from jax.experimental.pallas import tpu_sc as plsc

_F32 = jnp.float32


def _dot(a, b):
    return jnp.dot(a, b, preferred_element_type=_F32)


def _dot_hi(a, b):
    return jnp.dot(a, b, preferred_element_type=_F32,
                   precision=jax.lax.Precision.HIGHEST)


def _mm_body(x_ref, w_ref, o_ref):
    o_ref[...] = _dot(x_ref[...], w_ref[...])


def _matmul(x, w, bn):
    n, k = x.shape
    m = w.shape[1]
    return pl.pallas_call(
        _mm_body,
        grid=(n // bn,),
        in_specs=[pl.BlockSpec((bn, k), lambda i: (i, 0)),
                  pl.BlockSpec((k, m), lambda i: (0, 0))],
        out_specs=pl.BlockSpec((bn, m), lambda i: (i, 0)),
        out_shape=jax.ShapeDtypeStruct((n, m), _F32),
    )(x, w)


def _enc_body(ea_ref, w_ref, b_ref, o_ref):
    o_ref[...] = _dot(ea_ref[...], w_ref[...]) + b_ref[...]


def _edge_enc(edge_attr, w_t, b, be):
    e_num, d = edge_attr.shape
    h = w_t.shape[1]
    return pl.pallas_call(
        _enc_body,
        grid=(e_num // be,),
        in_specs=[pl.BlockSpec((be, d), lambda i: (i, 0)),
                  pl.BlockSpec((d, h), lambda i: (0, 0)),
                  pl.BlockSpec((1, h), lambda i: (0, 0))],
        out_specs=pl.BlockSpec((be, h), lambda i: (i, 0)),
        out_shape=jax.ShapeDtypeStruct((e_num, h), _F32),
    )(edge_attr, w_t, b)


def _pass_a_body(xr_ref, xc_ref, e_ref, w1rc_ref, we1_ref, b1_ref, w2_ref,
                 b2_ref, wn_ref, bn_ref, wn2_ref, bn2_ref, eo_ref, m_ref):
    xr = xr_ref[...]
    cat_rc = jnp.concatenate([xr, xc_ref[...]], axis=1)
    h1 = jnp.maximum(_dot(cat_rc, w1rc_ref[...])
                     + _dot(e_ref[...], we1_ref[...]) + b1_ref[...], 0.0)
    e2 = _dot(h1, w2_ref[...]) + b2_ref[...]
    eo_ref[...] = e2
    cat_n = jnp.concatenate([xr, e2], axis=1)
    h2 = jnp.maximum(_dot(cat_n, wn_ref[...]) + bn_ref[...], 0.0)
    m_ref[...] = _dot(h2, wn2_ref[...]) + bn2_ref[...]


def _pass_a(xr, xc, e, w1rc_t, we1_t, b1, w2_t, b2, wn_t, bn, wn2_t, bn2,
            be):
    e_num, h = e.shape
    kern = pl.pallas_call(
        _pass_a_body,
        grid=(e_num // be,),
        in_specs=[pl.BlockSpec((be, h), lambda i: (i, 0)),
                  pl.BlockSpec((be, h), lambda i: (i, 0)),
                  pl.BlockSpec((be, h), lambda i: (i, 0)),
                  pl.BlockSpec((2 * h, h), lambda i: (0, 0)),
                  pl.BlockSpec((h, h), lambda i: (0, 0)),
                  pl.BlockSpec((1, h), lambda i: (0, 0)),
                  pl.BlockSpec((h, h), lambda i: (0, 0)),
                  pl.BlockSpec((1, h), lambda i: (0, 0)),
                  pl.BlockSpec((2 * h, h), lambda i: (0, 0)),
                  pl.BlockSpec((1, h), lambda i: (0, 0)),
                  pl.BlockSpec((h, h), lambda i: (0, 0)),
                  pl.BlockSpec((1, h), lambda i: (0, 0))],
        out_specs=[pl.BlockSpec((be, h), lambda i: (i, 0)),
                   pl.BlockSpec((be, h), lambda i: (i, 0))],
        out_shape=[jax.ShapeDtypeStruct((e_num, h), _F32),
                   jax.ShapeDtypeStruct((e_num, h), _F32)],
    )
    return kern(xr, xc, e, w1rc_t, we1_t, b1, w2_t, b2, wn_t, bn, wn2_t, bn2)


def _pass_b_body(gd_ref, e_ref, w_ref, b_ref, w2_ref, b2_ref, o_ref):
    cat = jnp.concatenate([gd_ref[...], e_ref[...]], axis=1)
    h4 = jnp.maximum(_dot(cat, w_ref[...]) + b_ref[...], 0.0)
    o_ref[...] = _dot(h4, w2_ref[...]) + b2_ref[...]


def _pass_b(gd, e, w_t, b, w2_t, b2, be):
    e_num, h = e.shape
    return pl.pallas_call(
        _pass_b_body,
        grid=(e_num // be,),
        in_specs=[pl.BlockSpec((be, h), lambda i: (i, 0)),
                  pl.BlockSpec((be, h), lambda i: (i, 0)),
                  pl.BlockSpec((2 * h, h), lambda i: (0, 0)),
                  pl.BlockSpec((1, h), lambda i: (0, 0)),
                  pl.BlockSpec((h, h), lambda i: (0, 0)),
                  pl.BlockSpec((1, h), lambda i: (0, 0))],
        out_specs=pl.BlockSpec((be, h), lambda i: (i, 0)),
        out_shape=jax.ShapeDtypeStruct((e_num, h), _F32),
    )(gd, e, w_t, b, w2_t, b2)


def _pass_c_body(ge_ref, gf_ref, e_ref, w1rc_ref, we1_ref, b1_ref, w2_ref,
                 b2_ref, o_ref):
    cat = jnp.concatenate([ge_ref[...], gf_ref[...]], axis=1)
    h1 = jnp.maximum(_dot(cat, w1rc_ref[...])
                     + _dot(e_ref[...], we1_ref[...]) + b1_ref[...], 0.0)
    o_ref[...] = _dot(h1, w2_ref[...]) + b2_ref[...]


def _pass_c(ge, gf, e, w1rc_t, we1_t, b1, w2_t, b2, be):
    e_num, h = e.shape
    return pl.pallas_call(
        _pass_c_body,
        grid=(e_num // be,),
        in_specs=[pl.BlockSpec((be, h), lambda i: (i, 0)),
                  pl.BlockSpec((be, h), lambda i: (i, 0)),
                  pl.BlockSpec((be, h), lambda i: (i, 0)),
                  pl.BlockSpec((2 * h, h), lambda i: (0, 0)),
                  pl.BlockSpec((h, h), lambda i: (0, 0)),
                  pl.BlockSpec((1, h), lambda i: (0, 0)),
                  pl.BlockSpec((h, h), lambda i: (0, 0)),
                  pl.BlockSpec((1, h), lambda i: (0, 0))],
        out_specs=pl.BlockSpec((be, h), lambda i: (i, 0)),
        out_shape=jax.ShapeDtypeStruct((e_num, h), _F32),
    )(ge, gf, e, w1rc_t, we1_t, b1, w2_t, b2)


def _pool_body(acc_ref, batch_ref, w1_ref, b1_ref, w2_ref, b2_ref,
               o_ref, sacc_ref, *, g_num, n_blocks):
    i = pl.program_id(0)

    @pl.when(i == 0)
    def _init():
        sacc_ref[...] = jnp.zeros_like(sacc_ref)

    bvals = batch_ref[0, 0, :]
    pad = sacc_ref.shape[0]
    oh = (bvals[:, None] == jax.lax.broadcasted_iota(jnp.int32, (1, pad), 1)
          ).astype(_F32)
    sacc_ref[...] += _dot_hi(oh.T, acc_ref[...])

    @pl.when(i == n_blocks - 1)
    def _final():
        hid = jnp.maximum(_dot(sacc_ref[:g_num], w1_ref[...]) + b1_ref[...],
                          0.0)
        o_ref[...] = _dot(hid, w2_ref[...]) + b2_ref[...]


def _pool_head(acc, batch, w1_t, b1, w2_t, b2, g_num, bn):
    n, h = acc.shape
    n_blocks = n // bn
    pad = 128
    batch3d = batch.reshape(n_blocks, 1, bn)
    body = functools.partial(_pool_body, g_num=g_num, n_blocks=n_blocks)
    return pl.pallas_call(
        body,
        grid=(n_blocks,),
        in_specs=[pl.BlockSpec((bn, h), lambda i: (i, 0)),
                  pl.BlockSpec((1, 1, bn), lambda i: (i, 0, 0)),
                  pl.BlockSpec((h, h), lambda i: (0, 0)),
                  pl.BlockSpec((1, h), lambda i: (0, 0)),
                  pl.BlockSpec((h, 1), lambda i: (0, 0)),
                  pl.BlockSpec((1, 1), lambda i: (0, 0))],
        out_specs=pl.BlockSpec((g_num, 1), lambda i: (0, 0)),
        out_shape=jax.ShapeDtypeStruct((g_num, 1), _F32),
        scratch_shapes=[pltpu.VMEM((pad, h), _F32)],
    )(acc, batch3d, w1_t, b1, w2_t, b2)


_NW = 32
_CH = 40
_KB = 5


def _sc_gather_multi(tables, idxs):
    e_num = idxs[0].shape[0]
    ew = e_num // _NW
    nchunks = ew // _CH
    ngroups = nchunks // _KB
    assert e_num % _NW == 0 and ew % (_CH * _KB) == 0
    nt = len(tables)
    widths = [t.shape[1] for t in tables]

    scratch = []
    for w in widths:
        scratch.append(pltpu.VMEM((ew,), jnp.int32))
        scratch.append(pltpu.VMEM((_KB, _CH, w), _F32))
        for _ in range(2 * _KB):
            scratch.append(pltpu.SemaphoreType.DMA)

    mesh = plsc.VectorSubcoreMesh(core_axis_name="c", subcore_axis_name="s")

    @functools.partial(
        pl.kernel, mesh=mesh,
        out_type=[jax.ShapeDtypeStruct((e_num, w), _F32) for w in widths],
        scratch_types=scratch,
    )
    def gather_kernel(*args):
        tabs = args[:nt]
        ixs = args[nt:2 * nt]
        outs = args[2 * nt:3 * nt]
        rest = list(args[3 * nt:])
        wid = lax.axis_index("s") * 2 + lax.axis_index("c")
        base = wid * ew

        for t in range(nt):
            idx_v = rest[0]
            bufs = rest[1]
            gsems = rest[2:2 + _KB]
            wsems = rest[2 + _KB:2 + 2 * _KB]
            rest = rest[2 + 2 * _KB:]
            tab, out = tabs[t], outs[t]

            pltpu.sync_copy(ixs[t].at[pl.ds(base, ew)], idx_v)

            def group(g, _, tab=tab, out=out, idx_v=idx_v, bufs=bufs,
                      gsems=gsems, wsems=wsems):
                for b in range(_KB):
                    j = g * _KB + b

                    @pl.when(g > 0)
                    def _free():
                        pltpu.make_async_copy(
                            bufs.at[b], out.at[pl.ds(base, _CH)],
                            wsems[b]).wait()

                    pltpu.async_copy(tab.at[idx_v.at[pl.ds(j * _CH, _CH)]],
                                     bufs.at[b], gsems[b])
                for b in range(_KB):
                    j = g * _KB + b
                    pltpu.make_async_copy(
                        tab.at[idx_v.at[pl.ds(j * _CH, _CH)]], bufs.at[b],
                        gsems[b]).wait()
                    pltpu.async_copy(bufs.at[b],
                                     out.at[pl.ds(base + j * _CH, _CH)],
                                     wsems[b])
                return 0

            lax.fori_loop(0, ngroups, group, 0)
            for b in range(_KB):
                pltpu.make_async_copy(bufs.at[b], out.at[pl.ds(base, _CH)],
                                      wsems[b]).wait()

    return gather_kernel(*tables, *idxs)


def _scatter_add(vals, idx, n):
    return jax.ops.segment_sum(vals, idx, num_segments=n)


def kernel(x, edge_attr, edge_index, batch,
           edge_enc_W, edge_enc_b,
           etn_edge_W1, etn_edge_b1, etn_edge_W2, etn_edge_b2,
           etn_node_W1, etn_node_b1, etn_node_W2, etn_node_b2,
           nte_node_W1, nte_node_b1, nte_node_W2, nte_node_b2,
           nte_edge_W1, nte_edge_b1, nte_edge_W2, nte_edge_b2,
           reg_W1, reg_b1, reg_W2, reg_b2):
    n, h = x.shape
    e_num = edge_attr.shape[0]
    n_layers = etn_edge_W1.shape[0]
    g_num = 64
    be = 4000
    bn = 2000
    row = edge_index[0]
    col = edge_index[1]

    rvec = lambda v: v.reshape(1, -1)

    e = _edge_enc(edge_attr, edge_enc_W.T, rvec(edge_enc_b), be)

    acc = x
    xr = xc = None
    for l in range(n_layers):
        if xr is None:
            xr, xc = _sc_gather_multi([acc, acc], [row, col])
        e, m2 = _pass_a(xr, xc, e,
                        etn_edge_W1[l, :, :2 * h].T,
                        etn_edge_W1[l, :, 2 * h:].T, rvec(etn_edge_b1[l]),
                        etn_edge_W2[l].T, rvec(etn_edge_b2[l]),
                        etn_node_W1[l].T, rvec(etn_node_b1[l]),
                        etn_node_W2[l].T, rvec(etn_node_b2[l]), be)
        acc = _scatter_add(m2, col, n)

        (gd,) = _sc_gather_multi([acc], [row])
        m4 = _pass_b(gd, e, nte_node_W1[l].T, rvec(nte_node_b1[l]),
                     nte_node_W2[l].T, rvec(nte_node_b2[l]), be)
        acc = _scatter_add(m4, col, n)

        if l < n_layers - 1:
            xr, xc = _sc_gather_multi([acc, acc], [row, col])
            e = _pass_c(xr, xc, e,
                        nte_edge_W1[l, :, :2 * h].T,
                        nte_edge_W1[l, :, 2 * h:].T, rvec(nte_edge_b1[l]),
                        nte_edge_W2[l].T, rvec(nte_edge_b2[l]), be)

    out = _pool_head(acc, batch, reg_W1.T, rvec(reg_b1),
                     reg_W2.T, rvec(reg_b2), g_num, bn)
    return out.reshape(-1)
